# baseline (speedup 1.0000x reference)
import jax
import jax.numpy as jnp
from jax import lax
from jax.experimental import pallas as pl
from jax.experimental.pallas import tpu as pltpu
from jax.experimental.pallas import tpu_sc as plsc

B = 1024
NF = 100000
D = 16
OUT = 16
K = 300

NW = 32
ROWS_PER_TILE = B // NW
L = 16
NVEC = NF // L
CHUNK = 10000
NCHUNK = NF // CHUNK
NBINS = 512
HISTP = 513
CAP = 6144
GCH = 64
UNR = 5
ONE_BITS = 0x3F800000
SENT = -2147483648

f32 = jnp.float32
i32 = jnp.int32


def _splat(x, dtype=i32):
    return jnp.full((L,), x, dtype)


def _scal(v):
    return jnp.max(v)


def _tec_body(x_hbm, t_hbm, out_hbm, row_buf, hist, cand_idx, cand_val,
              trows, trows_b, out_stage, s0, s1, sg0, sg1, g0, g1):
    cid = lax.axis_index("c")
    sid = lax.axis_index("s")
    wid = sid * 2 + cid
    lane = lax.iota(i32, L)
    laneoff = lane * HISTP - (ONE_BITS >> 14)
    zeros_i = jnp.zeros((L,), i32)
    ones_i = jnp.ones((L,), i32)
    zf = jnp.zeros((L,), f32)
    capv = _splat(CAP)

    def per_row(rr, _carry):
        r = wid * ROWS_PER_TILE + rr

        sems = [s0, s1, sg0, sg1]
        cps = [pltpu.make_async_copy(
            x_hbm.at[r, pl.ds(c * CHUNK, CHUNK)],
            row_buf.at[pl.ds(c * CHUNK, CHUNK)],
            sems[c % 4]) for c in range(NCHUNK)]
        for c in range(4):
            cps[c].start()

        def clr(i, _):
            hist[pl.ds(i * L, L)] = zeros_i
            return 0
        lax.fori_loop(0, HISTP, clr, 0)

        for c in range(NCHUNK):
            cps[c].wait()
            if c + 4 < NCHUNK:
                cps[c + 4].start()
            base = c * CHUNK

            def s1body(i, _, base=base):
                for u in range(UNR):
                    x = row_buf[pl.ds(base + (i * UNR + u) * L, L)]
                    kb = plsc.bitcast(x + 1.0, i32) >> 14
                    plsc.addupdate_scatter(hist, [laneoff + kb], ones_i)
                return 0
            lax.fori_loop(0, CHUNK // L // UNR, s1body, 0)

        def sweep(j, carry):
            cum, found, b1 = carry
            jj = (NBINS // L - 1) - j
            tot = zeros_i
            for l in range(L):
                tot = tot + hist[pl.ds(jj * L + l * HISTP, L)]
            rv = lax.rev(tot, (0,))
            cs = plsc.cumsum(rv)
            full = cs + _splat(cum)
            mf = full >= K
            anyf = _scal(mf.astype(i32)) > 0
            fl = jnp.minimum(_scal(plsc.all_reduce_ffs(mf)), L - 1)
            b1_new = jj * L + (L - 1) - fl
            take = jnp.logical_and(jnp.logical_not(found), anyf)
            b1 = jnp.where(take, b1_new, b1)
            found = jnp.logical_or(found, anyf)
            cum = cum + jnp.sum(tot)
            return cum, found, b1

        _, _, b1 = lax.fori_loop(0, NBINS // L, sweep,
                                 (i32(0), False, i32(0)))

        tbv = _splat(ONE_BITS + (b1 << 14))

        def s2body(i, pos):
            for u in range(UNR):
                x = row_buf[pl.ds((i * UNR + u) * L, L)]
                yb = plsc.bitcast(x + 1.0, i32)
                m = yb >= tbv
                cs = plsc.cumsum(m.astype(i32))
                dest = pos + cs - 1
                ms = jnp.logical_and(m, dest < capv)
                iv = _splat((i * UNR + u) * L) + lane
                plsc.store_scatter(cand_idx, [dest], iv, mask=ms)
                plsc.store_scatter(cand_val, [dest],
                                   plsc.bitcast(x, i32), mask=ms)
                pos = pos + plsc.all_reduce_population_count(m)
            return pos
        pos = lax.fori_loop(0, NVEC // UNR, s2body, zeros_i)
        c = jnp.minimum(_scal(pos), CAP)

        for j in range(16):
            pdest = _splat(c + j * L) + lane
            plsc.store_scatter(cand_val, [pdest], _splat(SENT))
            plsc.store_scatter(cand_idx, [pdest], zeros_i)

        nv = (c + L - 1) >> 4

        def count_ge_c(u):
            uv = _splat(u)

            def cb(i, acc):
                bits = cand_val[pl.ds(i * L, L)]
                return acc + plsc.all_reduce_population_count(bits >= uv)
            return _scal(lax.fori_loop(0, nv, cb, zeros_i))

        def bstep(_i, lohi):
            lo, hi = lohi
            mid = lo + ((hi - lo + 1) >> 1)
            ge = count_ge_c(mid)
            lo = jnp.where(ge >= K, mid, lo)
            hi = jnp.where(ge >= K, hi, mid - 1)
            return lo, hi
        t, _ = lax.fori_loop(0, 31, bstep, (i32(0), i32(ONE_BITS)))
        g = count_ge_c(t + 1)
        mt = _splat(K - g)
        tv = _splat(t)

        nh = (c + 2 * GCH - 1) >> 7

        def gstart(ch, buf, sem):
            pltpu.make_async_copy(
                t_hbm.at[cand_idx.at[pl.ds(ch * GCH, GCH)]],
                buf, sem).start()

        def gwait(buf, sem):
            pltpu.make_async_copy(t_hbm.at[cand_idx.at[pl.ds(0, GCH)]],
                                  buf, sem).wait()

        gstart(i32(0), trows, g0)
        gstart(i32(1), trows_b, g1)

        def consume(buf, base, carry):
            ae, aq, al, tec = carry
            for v4 in range(GCH // L):
                bits = cand_val[pl.ds(base + v4 * L, L)]
                m_gt = bits > tv
                m_eq = bits == tv
                rank = tec + plsc.cumsum(m_eq.astype(i32))
                keep = jnp.logical_or(m_gt,
                                      jnp.logical_and(m_eq, rank <= mt))
                tec = tec + plsc.all_reduce_population_count(m_eq)
                w = jnp.where(keep, plsc.bitcast(bits, f32), 0.0)
                for i in range(L):
                    wb = jnp.sum(jnp.where(lane == i, w, 0.0))
                    t0 = buf[v4 * L + i, pl.ds(0, L)]
                    t1 = buf[v4 * L + i, pl.ds(L, L)]
                    p = wb * t0
                    ae = ae + p
                    aq = aq + p * p
                    al = al + wb * t1
            return ae, aq, al, tec

        def acc_body(h, carry):
            ch = 2 * h
            gwait(trows, g0)
            carry = consume(trows, ch * GCH, carry)
            gstart(ch + 2, trows, g0)
            gwait(trows_b, g1)
            carry = consume(trows_b, (ch + 1) * GCH, carry)
            gstart(ch + 3, trows_b, g1)
            return carry

        ae, aq, al, _ = lax.fori_loop(0, nh, acc_body,
                                      (zf, zf, zf, zeros_i))
        gwait(trows, g0)
        gwait(trows_b, g1)

        out_stage[pl.ds(0, L)] = ae
        out_stage[pl.ds(L, L)] = al
        out_stage[pl.ds(2 * L, L)] = aq
        out_stage[pl.ds(3 * L, L)] = zf
        pltpu.sync_copy(out_stage, out_hbm.at[r])
        return 0

    lax.fori_loop(0, ROWS_PER_TILE, per_row, 0)


def _sc_topk_accum(x, t_tab):
    mesh = plsc.VectorSubcoreMesh(core_axis_name="c", subcore_axis_name="s")
    fn = pl.kernel(
        _tec_body,
        out_type=jax.ShapeDtypeStruct((B, 4 * L), f32),
        mesh=mesh,
        compiler_params=pltpu.CompilerParams(use_tc_tiling_on_sc=False,
                                             needs_layout_passes=False),
        scratch_types=[
            pltpu.VMEM((NF,), f32),
            pltpu.VMEM((L * HISTP,), i32),
            pltpu.VMEM((CAP + 320,), i32),
            pltpu.VMEM((CAP + 320,), i32),
            pltpu.VMEM((GCH, 2 * L), f32),
            pltpu.VMEM((GCH, 2 * L), f32),
            pltpu.VMEM((4 * L,), f32),
            pltpu.SemaphoreType.DMA,
            pltpu.SemaphoreType.DMA,
            pltpu.SemaphoreType.DMA,
            pltpu.SemaphoreType.DMA,
            pltpu.SemaphoreType.DMA,
            pltpu.SemaphoreType.DMA,
        ],
    )
    return fn(x, t_tab)


def _head_body(s_ref, lb_ref, w1t_ref, b1_ref, w2t_ref, b2_ref,
               out_ref, lin_ref, int_ref):
    s = s_ref[...]
    se = s[:, 0:OUT]
    sl = s[:, OUT:2 * OUT]
    sq = s[:, 2 * OUT:3 * OUT]
    iv = 0.5 * (se * se - sq)
    h = jnp.maximum(
        jnp.dot(iv, w1t_ref[...], preferred_element_type=f32) + b1_ref[...],
        0.0)
    io = jnp.dot(h, w2t_ref[...], preferred_element_type=f32) + b2_ref[...]
    lo = sl + lb_ref[...]
    lin_ref[...] = lo
    int_ref[...] = io
    out_ref[...] = lo + io


def kernel(sae_features, emb, lin_W, lin_b, W1, b1, W2, b2):
    t_tab = jnp.concatenate([emb, lin_W.T], axis=1)
    sums = _sc_topk_accum(sae_features, t_tab)
    outs = pl.pallas_call(
        _head_body,
        out_shape=[jax.ShapeDtypeStruct((B, OUT), f32)] * 3,
    )(sums, lin_b.reshape(1, OUT), W1.T, b1.reshape(1, OUT),
      W2.T, b2.reshape(1, OUT))
    output, linear_out, interaction_out = outs
    return (output, linear_out, interaction_out)


# speedup vs baseline: 1.0277x; 1.0277x over previous
import jax
import jax.numpy as jnp
from jax import lax
from jax.experimental import pallas as pl
from jax.experimental.pallas import tpu as pltpu
from jax.experimental.pallas import tpu_sc as plsc

B = 1024
NF = 100000
D = 16
OUT = 16
K = 300

NW = 32
ROWS_PER_TILE = B // NW
L = 16
NVEC = NF // L
CHUNK = 10000
NCHUNK = NF // CHUNK
NBINS = 512
HISTP = NBINS
BINF = 448
CAP = 6144
GCH = 64
UNR = 5
ONE_BITS = 0x3F800000
SENT = -2147483648

f32 = jnp.float32
i32 = jnp.int32


def _splat(x, dtype=i32):
    return jnp.full((L,), x, dtype)


def _scal(v):
    return jnp.max(v)


def _tec_body(x_hbm, t_hbm, out_hbm, row_buf, hist, cand_idx, cand_val,
              trows, trows_b, out_stage, s0, s1, sg0, sg1, g0, g1):
    cid = lax.axis_index("c")
    sid = lax.axis_index("s")
    wid = sid * 2 + cid
    lane = lax.iota(i32, L)
    laneoff = lane * HISTP - (ONE_BITS >> 14)
    zeros_i = jnp.zeros((L,), i32)
    ones_i = jnp.ones((L,), i32)
    zf = jnp.zeros((L,), f32)
    capv = _splat(CAP)
    bfv = _splat((ONE_BITS >> 14) + BINF)

    def per_row(rr, _carry):
        r = wid * ROWS_PER_TILE + rr

        sems = [s0, s1, sg0, sg1]
        cps = [pltpu.make_async_copy(
            x_hbm.at[r, pl.ds(c * CHUNK, CHUNK)],
            row_buf.at[pl.ds(c * CHUNK, CHUNK)],
            sems[c % 4]) for c in range(NCHUNK)]
        for c in range(4):
            cps[c].start()

        def clr(i, _):
            for u in range(4):
                hist[pl.ds((i * 4 + u) * L, L)] = zeros_i
            return 0
        lax.fori_loop(0, HISTP // 4, clr, 0)

        cnt_hi = zeros_i
        for c in range(NCHUNK):
            cps[c].wait()
            if c + 4 < NCHUNK:
                cps[c + 4].start()
            base = c * CHUNK

            def s1body(i, cnt, base=base):
                for u in range(UNR):
                    x = row_buf[pl.ds(base + (i * UNR + u) * L, L)]
                    kb = plsc.bitcast(x + 1.0, i32) >> 14
                    m = kb >= bfv
                    plsc.addupdate_scatter(hist, [laneoff + kb], ones_i,
                                           mask=m)
                    cnt = cnt + plsc.all_reduce_population_count(m)
                return cnt
            cnt_hi = lax.fori_loop(0, CHUNK // L // UNR, s1body, cnt_hi)

        @pl.when(_scal(cnt_hi) < K)
        def _fallback_fill():
            def fb(i, _):
                x = row_buf[pl.ds(i * L, L)]
                kb = plsc.bitcast(x + 1.0, i32) >> 14
                plsc.addupdate_scatter(hist, [laneoff + kb], ones_i,
                                       mask=kb < bfv)
                return 0
            lax.fori_loop(0, NVEC, fb, 0)

        def sweep(j, carry):
            cum, found, b1 = carry
            jj = (NBINS // L - 1) - j
            tot = zeros_i
            for l in range(L):
                tot = tot + hist[pl.ds(jj * L + l * HISTP, L)]
            rv = lax.rev(tot, (0,))
            cs = plsc.cumsum(rv)
            full = cs + _splat(cum)
            mf = full >= K
            anyf = _scal(mf.astype(i32)) > 0
            fl = jnp.minimum(_scal(plsc.all_reduce_ffs(mf)), L - 1)
            b1_new = jj * L + (L - 1) - fl
            take = jnp.logical_and(jnp.logical_not(found), anyf)
            b1 = jnp.where(take, b1_new, b1)
            found = jnp.logical_or(found, anyf)
            cum = cum + jnp.sum(tot)
            return cum, found, b1

        _, _, b1 = lax.fori_loop(0, NBINS // L, sweep,
                                 (i32(0), False, i32(0)))

        tbv = _splat(ONE_BITS + (b1 << 14))

        def s2body(i, pos):
            for u in range(UNR):
                x = row_buf[pl.ds((i * UNR + u) * L, L)]
                yb = plsc.bitcast(x + 1.0, i32)
                m = yb >= tbv
                cs = plsc.cumsum(m.astype(i32))
                dest = pos + cs - 1
                ms = jnp.logical_and(m, dest < capv)
                iv = _splat((i * UNR + u) * L) + lane
                plsc.store_scatter(cand_idx, [dest], iv, mask=ms)
                plsc.store_scatter(cand_val, [dest],
                                   plsc.bitcast(x, i32), mask=ms)
                pos = pos + plsc.all_reduce_population_count(m)
            return pos
        pos = lax.fori_loop(0, NVEC // UNR, s2body, zeros_i)
        c = jnp.minimum(_scal(pos), CAP)

        for j in range(16):
            pdest = _splat(c + j * L) + lane
            plsc.store_scatter(cand_val, [pdest], _splat(SENT))
            plsc.store_scatter(cand_idx, [pdest], zeros_i)

        nv = (c + L - 1) >> 4

        def count_ge_c(u):
            uv = _splat(u)

            def cb(i, acc):
                bits = cand_val[pl.ds(i * L, L)]
                return acc + plsc.all_reduce_population_count(bits >= uv)
            return _scal(lax.fori_loop(0, nv, cb, zeros_i))

        def bstep(_i, lohi):
            lo, hi = lohi
            mid = lo + ((hi - lo + 1) >> 1)
            ge = count_ge_c(mid)
            lo = jnp.where(ge >= K, mid, lo)
            hi = jnp.where(ge >= K, hi, mid - 1)
            return lo, hi
        t, _ = lax.fori_loop(0, 31, bstep, (i32(0), i32(ONE_BITS)))
        g = count_ge_c(t + 1)
        mt = _splat(K - g)
        tv = _splat(t)

        nch = (c + GCH - 1) >> 6

        def acc_body(ch, carry):
            ae, aq, al, tec = carry
            cp = pltpu.make_async_copy(
                t_hbm.at[cand_idx.at[pl.ds(ch * GCH, GCH)]], trows, g0)
            cp.start()
            cp.wait()
            for v4 in range(GCH // L):
                bits = cand_val[pl.ds(ch * GCH + v4 * L, L)]
                m_gt = bits > tv
                m_eq = bits == tv
                rank = tec + plsc.cumsum(m_eq.astype(i32))
                keep = jnp.logical_or(m_gt,
                                      jnp.logical_and(m_eq, rank <= mt))
                tec = tec + plsc.all_reduce_population_count(m_eq)
                w = jnp.where(keep, plsc.bitcast(bits, f32), 0.0)
                for i in range(L):
                    wb = jnp.sum(jnp.where(lane == i, w, 0.0))
                    t0 = trows[v4 * L + i, pl.ds(0, L)]
                    t1 = trows[v4 * L + i, pl.ds(L, L)]
                    p = wb * t0
                    ae = ae + p
                    aq = aq + p * p
                    al = al + wb * t1
            return ae, aq, al, tec

        ae, aq, al, _ = lax.fori_loop(0, nch, acc_body,
                                      (zf, zf, zf, zeros_i))

        out_stage[pl.ds(0, L)] = ae
        out_stage[pl.ds(L, L)] = al
        out_stage[pl.ds(2 * L, L)] = aq
        out_stage[pl.ds(3 * L, L)] = zf
        pltpu.sync_copy(out_stage, out_hbm.at[r])
        return 0

    lax.fori_loop(0, ROWS_PER_TILE, per_row, 0)


def _sc_topk_accum(x, t_tab):
    mesh = plsc.VectorSubcoreMesh(core_axis_name="c", subcore_axis_name="s")
    fn = pl.kernel(
        _tec_body,
        out_type=jax.ShapeDtypeStruct((B, 4 * L), f32),
        mesh=mesh,
        compiler_params=pltpu.CompilerParams(use_tc_tiling_on_sc=False,
                                             needs_layout_passes=False),
        scratch_types=[
            pltpu.VMEM((NF,), f32),
            pltpu.VMEM((L * HISTP,), i32),
            pltpu.VMEM((CAP + 320,), i32),
            pltpu.VMEM((CAP + 320,), i32),
            pltpu.VMEM((GCH, 2 * L), f32),
            pltpu.VMEM((GCH, 2 * L), f32),
            pltpu.VMEM((4 * L,), f32),
            pltpu.SemaphoreType.DMA,
            pltpu.SemaphoreType.DMA,
            pltpu.SemaphoreType.DMA,
            pltpu.SemaphoreType.DMA,
            pltpu.SemaphoreType.DMA,
            pltpu.SemaphoreType.DMA,
        ],
    )
    return fn(x, t_tab)


def _head_body(s_ref, lb_ref, w1t_ref, b1_ref, w2t_ref, b2_ref,
               out_ref, lin_ref, int_ref):
    s = s_ref[...]
    se = s[:, 0:OUT]
    sl = s[:, OUT:2 * OUT]
    sq = s[:, 2 * OUT:3 * OUT]
    iv = 0.5 * (se * se - sq)
    h = jnp.maximum(
        jnp.dot(iv, w1t_ref[...], preferred_element_type=f32) + b1_ref[...],
        0.0)
    io = jnp.dot(h, w2t_ref[...], preferred_element_type=f32) + b2_ref[...]
    lo = sl + lb_ref[...]
    lin_ref[...] = lo
    int_ref[...] = io
    out_ref[...] = lo + io


def kernel(sae_features, emb, lin_W, lin_b, W1, b1, W2, b2):
    t_tab = jnp.concatenate([emb, lin_W.T], axis=1)
    sums = _sc_topk_accum(sae_features, t_tab)
    outs = pl.pallas_call(
        _head_body,
        out_shape=[jax.ShapeDtypeStruct((B, OUT), f32)] * 3,
    )(sums, lin_b.reshape(1, OUT), W1.T, b1.reshape(1, OUT),
      W2.T, b2.reshape(1, OUT))
    output, linear_out, interaction_out = outs
    return (output, linear_out, interaction_out)
